# Initial kernel scaffold; baseline (speedup 1.0000x reference)
import jax
import jax.numpy as jnp
from jax.experimental import pallas as pl


def kernel(x, log_alpha, log_cp_cores, log_pi_ref_table):
    raise NotImplementedError("write your pallas kernel here")


# speedup vs baseline: 10.8179x; 10.8179x over previous
import jax
import jax.numpy as jnp
from jax.experimental import pallas as pl


_HI = jax.lax.Precision.HIGHEST


def _select_body(x_ref, coresT_ref, table_ref, la_ref, gk_ref, ohk_ref):
    B, D = x_ref.shape
    C, K = coresT_ref.shape[1], coresT_ref.shape[2]
    ET = jnp.exp(table_ref[...])
    iota_c = jax.lax.broadcasted_iota(jnp.int32, (B, C), 1)
    acc = jnp.zeros((B, K), dtype=jnp.float32)
    for d in range(D):
        Pd = jnp.exp(coresT_ref[d])
        Md = jnp.dot(ET, Pd, precision=_HI, preferred_element_type=jnp.float32)
        Ld = jnp.log(Md)
        oh = (x_ref[:, d:d + 1] == iota_c).astype(jnp.float32)
        acc = acc + jnp.dot(oh, Ld, precision=_HI,
                            preferred_element_type=jnp.float32)
    log_w = la_ref[...] + acc
    mw = jnp.max(log_w, axis=1, keepdims=True)
    lse = jnp.log(jnp.sum(jnp.exp(log_w - mw), axis=1, keepdims=True)) + mw
    score = (log_w - lse) + gk_ref[...]
    m = jnp.max(score, axis=1, keepdims=True)
    iota_k = jax.lax.broadcasted_iota(jnp.int32, (B, K), 1)
    idx = jnp.where(score == m, iota_k, K)
    kmin = jnp.min(idx, axis=1, keepdims=True)
    ohk_ref[...] = (iota_k == kmin).astype(jnp.float32)


def _sample_body(cores_ref, table_ref, ohk_ref, gy_ref, x3_ref, y3_ref):
    B = ohk_ref.shape[0]
    C = table_ref.shape[0]
    xcol = x3_ref[0]
    iota_c = jax.lax.broadcasted_iota(jnp.int32, (B, C), 1)
    ohx = (xcol == iota_c).astype(jnp.float32)
    pi = jnp.dot(ohx, table_ref[...], precision=_HI,
                 preferred_element_type=jnp.float32)
    rows = jnp.dot(ohk_ref[...], cores_ref[0], precision=_HI,
                   preferred_element_type=jnp.float32)
    sel = rows + pi + gy_ref[0]
    m = jnp.max(sel, axis=1, keepdims=True)
    idx = jnp.where(sel == m, iota_c, C)
    y = jnp.min(idx, axis=1, keepdims=True)
    y3_ref[...] = y[None]


def kernel(x, log_alpha, log_cp_cores, log_pi_ref_table):
    B, D = x.shape
    K = log_alpha.shape[0]
    C = log_pi_ref_table.shape[0]

    skey = jax.random.key(42)
    k_key, y_key = jax.random.split(skey)
    g_k = jax.random.gumbel(k_key, (B, K), jnp.float32)
    y_keys = jax.random.split(y_key, D)
    g_y = jax.vmap(lambda kk: jax.random.gumbel(kk, (B, C), jnp.float32))(y_keys)

    coresT = jnp.transpose(log_cp_cores, (0, 2, 1))
    la = log_alpha.reshape(1, K)
    x3 = x.T.reshape(D, B, 1)

    ohk = pl.pallas_call(
        _select_body,
        out_shape=jax.ShapeDtypeStruct((B, K), jnp.float32),
    )(x, coresT, log_pi_ref_table, la, g_k)

    y3 = pl.pallas_call(
        _sample_body,
        grid=(D,),
        in_specs=[
            pl.BlockSpec((1, K, C), lambda d: (d, 0, 0)),
            pl.BlockSpec((C, C), lambda d: (0, 0)),
            pl.BlockSpec((B, K), lambda d: (0, 0)),
            pl.BlockSpec((1, B, C), lambda d: (d, 0, 0)),
            pl.BlockSpec((1, B, 1), lambda d: (d, 0, 0)),
        ],
        out_specs=pl.BlockSpec((1, B, 1), lambda d: (d, 0, 0)),
        out_shape=jax.ShapeDtypeStruct((D, B, 1), jnp.int32),
    )(log_cp_cores, log_pi_ref_table, ohk, g_y, x3)

    return y3.reshape(D, B).T


# speedup vs baseline: 12.4467x; 1.1506x over previous
import jax
import jax.numpy as jnp
from jax.experimental import pallas as pl


_HI = jax.lax.Precision.HIGHEST


def _select_body(x_ref, coresT_ref, table_ref, la_ref, gk_ref, ohk_ref):
    B, D = x_ref.shape
    C, K = coresT_ref.shape[1], coresT_ref.shape[2]
    ET = jnp.exp(table_ref[...])
    iota_c = jax.lax.broadcasted_iota(jnp.int32, (B, C), 1)
    acc = jnp.zeros((B, K), dtype=jnp.float32)
    for d in range(D):
        Pd = jnp.exp(coresT_ref[d])
        Md = jnp.dot(ET, Pd, precision=_HI, preferred_element_type=jnp.float32)
        Ld = jnp.log(Md)
        oh = (x_ref[:, d:d + 1] == iota_c).astype(jnp.float32)
        acc = acc + jnp.dot(oh, Ld, precision=_HI,
                            preferred_element_type=jnp.float32)
    log_w = la_ref[...] + acc
    mw = jnp.max(log_w, axis=1, keepdims=True)
    lse = jnp.log(jnp.sum(jnp.exp(log_w - mw), axis=1, keepdims=True)) + mw
    score = (log_w - lse) + gk_ref[...]
    m = jnp.max(score, axis=1, keepdims=True)
    iota_k = jax.lax.broadcasted_iota(jnp.int32, (B, K), 1)
    idx = jnp.where(score == m, iota_k, K)
    kmin = jnp.min(idx, axis=1, keepdims=True)
    ohk_ref[...] = (iota_k == kmin).astype(jnp.float32)


def _sample_body(cores_ref, table_ref, ohk_ref, gy_ref, x3_ref, y3_ref):
    B = ohk_ref.shape[0]
    C = table_ref.shape[0]
    xcol = x3_ref[0]
    iota_c = jax.lax.broadcasted_iota(jnp.int32, (B, C), 1)
    ohx = (xcol == iota_c).astype(jnp.float32)
    pi = jnp.dot(ohx, table_ref[...], precision=_HI,
                 preferred_element_type=jnp.float32)
    rows = jnp.dot(ohk_ref[...], cores_ref[0], precision=_HI,
                   preferred_element_type=jnp.float32)
    sel = rows + pi + gy_ref[0]
    m = jnp.max(sel, axis=1, keepdims=True)
    idx = jnp.where(sel == m, iota_c, C)
    y = jnp.min(idx, axis=1, keepdims=True)
    y3_ref[...] = y[None]


def kernel(x, log_alpha, log_cp_cores, log_pi_ref_table):
    B, D = x.shape
    K = log_alpha.shape[0]
    C = log_pi_ref_table.shape[0]

    skey = jax.random.key(42)
    k_key, y_key = jax.random.split(skey)
    g_k = jax.random.gumbel(k_key, (B, K), jnp.float32)
    y_keys = jax.random.split(y_key, D)
    g_y = jax.vmap(lambda kk: jax.random.gumbel(kk, (B, C), jnp.float32))(y_keys)

    coresT = jnp.transpose(log_cp_cores, (0, 2, 1))
    la = log_alpha.reshape(1, K)
    x3 = x.T.reshape(D, B, 1)

    import os as _os
    _variant = _os.environ.get("KVARIANT", "full")

    if _variant == "gumbel_only":
        def _mini(gy_ref, y3_ref):
            sel = gy_ref[0]
            m = jnp.max(sel, axis=1, keepdims=True)
            iota_c = jax.lax.broadcasted_iota(jnp.int32, sel.shape, 1)
            idx = jnp.where(sel == m, iota_c, sel.shape[1])
            y3_ref[...] = jnp.min(idx, axis=1, keepdims=True)[None]
        y3 = pl.pallas_call(
            _mini,
            grid=(D,),
            in_specs=[pl.BlockSpec((1, B, C), lambda d: (d, 0, 0))],
            out_specs=pl.BlockSpec((1, B, 1), lambda d: (d, 0, 0)),
            out_shape=jax.ShapeDtypeStruct((D, B, 1), jnp.int32),
        )(g_y + g_k.sum())
        return y3.reshape(D, B).T

    ohk = pl.pallas_call(
        _select_body,
        out_shape=jax.ShapeDtypeStruct((B, K), jnp.float32),
    )(x, coresT, log_pi_ref_table, la, g_k)

    if _variant == "select_only":
        def _mini2(gy_ref, ohk_ref, y3_ref):
            sel = gy_ref[0] + ohk_ref[0:1, 0:1]
            m = jnp.max(sel, axis=1, keepdims=True)
            iota_c = jax.lax.broadcasted_iota(jnp.int32, sel.shape, 1)
            idx = jnp.where(sel == m, iota_c, sel.shape[1])
            y3_ref[...] = jnp.min(idx, axis=1, keepdims=True)[None]
        y3 = pl.pallas_call(
            _mini2,
            grid=(D,),
            in_specs=[pl.BlockSpec((1, B, C), lambda d: (d, 0, 0)),
                      pl.BlockSpec((B, K), lambda d: (0, 0))],
            out_specs=pl.BlockSpec((1, B, 1), lambda d: (d, 0, 0)),
            out_shape=jax.ShapeDtypeStruct((D, B, 1), jnp.int32),
        )(g_y, ohk)
        return y3.reshape(D, B).T

    y3 = pl.pallas_call(
        _sample_body,
        grid=(D,),
        in_specs=[
            pl.BlockSpec((1, K, C), lambda d: (d, 0, 0)),
            pl.BlockSpec((C, C), lambda d: (0, 0)),
            pl.BlockSpec((B, K), lambda d: (0, 0)),
            pl.BlockSpec((1, B, C), lambda d: (d, 0, 0)),
            pl.BlockSpec((1, B, 1), lambda d: (d, 0, 0)),
        ],
        out_specs=pl.BlockSpec((1, B, 1), lambda d: (d, 0, 0)),
        out_shape=jax.ShapeDtypeStruct((D, B, 1), jnp.int32),
    )(log_cp_cores, log_pi_ref_table, ohk, g_y, x3)

    return y3.reshape(D, B).T
